# baseline (speedup 1.0000x reference)
import functools

import jax
import jax.numpy as jnp
from jax import lax
from jax.experimental import pallas as pl
from jax.experimental.pallas import tpu as pltpu
from jax.experimental.pallas import tpu_sc as plsc

_D = 64
_B = 4096
_L = 50
_HID = 100
_NCLS = 4
_HPAD = 128

_NC = 2
_NS = 16
_NW = _NC * _NS
_BPW = _B // _NW
_EPG = 2
_GROUP = _EPG * _L
_NG = _BPW // _EPG


def _pool_sc(xg, table):
    mesh = plsc.VectorSubcoreMesh(core_axis_name="c", subcore_axis_name="s")

    @functools.partial(
        pl.kernel,
        out_type=jax.ShapeDtypeStruct((_B, _D), jnp.float32),
        mesh=mesh,
        compiler_params=pltpu.CompilerParams(use_tc_tiling_on_sc=False),
        scratch_types=[
            pltpu.VMEM((_NG, _GROUP), jnp.int32),
            pltpu.VMEM((_GROUP, _D), jnp.float32),
            pltpu.VMEM((_GROUP, _D), jnp.float32),
            pltpu.VMEM((_BPW, _D), jnp.float32),
            pltpu.SemaphoreType.DMA,
            pltpu.SemaphoreType.DMA,
        ],
    )
    def k(xg_hbm, tbl_hbm, out_hbm, idx_v, rows0, rows1, pooled_v, sem0, sem1):
        wid = lax.axis_index("s") * _NC + lax.axis_index("c")
        pltpu.sync_copy(xg_hbm.at[wid], idx_v)

        def fire(g, rows, sem):
            pltpu.async_copy(tbl_hbm.at[idx_v.at[g]], rows, sem)

        def wait(g, rows, sem):
            pltpu.make_async_copy(tbl_hbm.at[idx_v.at[g]], rows, sem).wait()

        def accum(g, rows):
            def body(r, accs):
                nxt = []
                for e in range(_EPG):
                    for d4 in range(4):
                        sl = pl.ds(16 * d4, 16)
                        nxt.append(accs[e * 4 + d4] + rows[e * _L + r, sl])
                return tuple(nxt)

            init = tuple(jnp.zeros((16,), jnp.float32)
                         for _ in range(_EPG * 4))
            accs = lax.fori_loop(0, _L, body, init, unroll=2)
            for e in range(_EPG):
                for d4 in range(4):
                    pooled_v[g * _EPG + e, pl.ds(16 * d4, 16)] = accs[e * 4 + d4]

        fire(0, rows0, sem0)
        fire(1, rows1, sem1)

        def gbody(i, _):
            g0 = 2 * i
            wait(g0, rows0, sem0)
            accum(g0, rows0)

            @pl.when(g0 + 2 < _NG)
            def _f0():
                fire(g0 + 2, rows0, sem0)

            wait(g0 + 1, rows1, sem1)
            accum(g0 + 1, rows1)

            @pl.when(g0 + 3 < _NG)
            def _f1():
                fire(g0 + 3, rows1, sem1)

            return 0

        lax.fori_loop(0, _NG // 2, gbody, 0)
        pltpu.sync_copy(pooled_v, out_hbm.at[pl.ds(wid * _BPW, _BPW)])

    return k(xg, table)


def _mlp_tc(pooled, w1p, b1p, w2p, b2p):
    def body(p_ref, w1_ref, b1_ref, w2_ref, b2_ref, o_ref):
        h = jnp.dot(p_ref[...], w1_ref[...],
                    preferred_element_type=jnp.float32)
        h = jnp.maximum(h + b1_ref[...], 0.0)
        o = jnp.dot(h, w2_ref[...], preferred_element_type=jnp.float32)
        o_ref[...] = jnp.maximum(o + b2_ref[...], 0.0)

    return pl.pallas_call(
        body,
        out_shape=jax.ShapeDtypeStruct((_B, _NCLS), jnp.float32),
    )(pooled, w1p, b1p, w2p, b2p)


def kernel(x, table, W1, b1, W2, b2):
    xg = x.reshape(_NW, _NG, _GROUP)
    pooled = _pool_sc(xg, table)
    w1p = jnp.pad(W1, ((0, 0), (0, _HPAD - _HID)))
    b1p = jnp.pad(b1, (0, _HPAD - _HID)).reshape(1, _HPAD)
    w2p = jnp.pad(W2, ((0, _HPAD - _HID), (0, 0)))
    b2p = b2.reshape(1, _NCLS)
    return _mlp_tc(pooled, w1p, b1p, w2p, b2p)
